# baseline (speedup 1.0000x reference)
import functools

import jax
import jax.numpy as jnp
from jax import lax
from jax.experimental import pallas as pl
from jax.experimental.pallas import tpu as pltpu
from jax.experimental.pallas import tpu_sc as plsc

_VOCAB = 100000
_HIDDEN = 128
_MAX_LEN = 2048
_BATCH = 4
_NC = 2
_NS = 16
_NW = _NC * _NS
_LANES = 16
_S_PER_W = _MAX_LEN // _NW
_TOK_PER_W = _BATCH * _S_PER_W
_JJ = _HIDDEN // _LANES


def _emb_kernel(ids_hbm, tt_hbm, word_hbm, pos_hbm, type_hbm, out_hbm,
                idx_v, tti_v, ttv_v, we_v, pe_v, ty_v,
                g_sems, out_sem, pe_sem):
    wid = lax.axis_index("s") * _NC + lax.axis_index("c")
    s0 = wid * _S_PER_W

    idx_copies = [
        pltpu.async_copy(ids_hbm.at[b, pl.ds(s0, _S_PER_W)],
                         idx_v.at[b // 2, pl.ds((b % 2) * _S_PER_W, _S_PER_W)],
                         g_sems.at[b // 2])
        for b in range(_BATCH)
    ]
    tt_copies = [
        pltpu.async_copy(tt_hbm.at[b, pl.ds(s0, _S_PER_W)], tti_v.at[b],
                         out_sem)
        for b in range(_BATCH)
    ]
    ty_copy = pltpu.async_copy(type_hbm, ty_v, out_sem)
    pe_copy = pltpu.async_copy(pos_hbm.at[pl.ds(s0, _S_PER_W)], pe_v, pe_sem)
    gathers = []
    for c in range(_BATCH // 2):
        idx_copies[2 * c].wait()
        idx_copies[2 * c + 1].wait()
        gathers.append(
            pltpu.async_copy(word_hbm.at[idx_v.at[c]],
                             we_v.at[pl.ds(c * 2 * _S_PER_W, 2 * _S_PER_W)],
                             g_sems.at[c]))
    for c in tt_copies:
        c.wait()
    ty_copy.wait()

    def splat(g, _):
        b = g // (_S_PER_W // _LANES)
        r0 = (g % (_S_PER_W // _LANES)) * _LANES
        tts = tti_v[b, pl.ds(r0, _LANES)].astype(jnp.float32)
        for k in range(_LANES):
            ttv_v[b * _S_PER_W + r0 + k, pl.ds(0, _LANES)] = jnp.full(
                (_LANES,), tts[k], jnp.float32)
        return _

    lax.fori_loop(0, _TOK_PER_W // _LANES, splat, None)

    t0 = [ty_v[0, pl.ds(j * _LANES, _LANES)] for j in range(_JJ)]
    td = [ty_v[1, pl.ds(j * _LANES, _LANES)] - t0[j] for j in range(_JJ)]
    pe_copy.wait()

    outs = []
    for b in range(_BATCH):
        if b % 2 == 0:
            gathers[b // 2].wait()

        def body(r, _, b=b):
            i = b * _S_PER_W + r
            ttv = ttv_v[i, pl.ds(0, _LANES)]
            for j in range(_JJ):
                c = j * _LANES
                we_v[i, pl.ds(c, _LANES)] = (
                    we_v[i, pl.ds(c, _LANES)] + pe_v[r, pl.ds(c, _LANES)]
                    + (t0[j] + ttv * td[j])
                )
            return _

        lax.fori_loop(0, _S_PER_W, body, None)
        outs.append(pltpu.async_copy(
            we_v.at[pl.ds(b * _S_PER_W, _S_PER_W)],
            out_hbm.at[pl.ds(b * _MAX_LEN + s0, _S_PER_W)],
            out_sem))
    for o in outs:
        o.wait()


@jax.jit
def _embedding_sum(ids, tt, word_emb, pos_emb, type_emb):
    mesh = plsc.VectorSubcoreMesh(core_axis_name="c", subcore_axis_name="s")
    kfn = functools.partial(
        pl.kernel,
        mesh=mesh,
        out_type=jax.ShapeDtypeStruct((_BATCH * _MAX_LEN, _HIDDEN), jnp.float32),
        scratch_types=[
            pltpu.VMEM((_BATCH // 2, 2 * _S_PER_W), jnp.int32),
            pltpu.VMEM((_BATCH, _S_PER_W), jnp.int32),
            pltpu.VMEM((_TOK_PER_W, _LANES), jnp.float32),
            pltpu.VMEM((_TOK_PER_W, _HIDDEN), jnp.float32),
            pltpu.VMEM((_S_PER_W, _HIDDEN), jnp.float32),
            pltpu.VMEM((2, _HIDDEN), jnp.float32),
            pltpu.SemaphoreType.DMA((_BATCH // 2,)),
            pltpu.SemaphoreType.DMA,
            pltpu.SemaphoreType.DMA,
        ],
    )(_emb_kernel)
    return kfn(ids, tt, word_emb, pos_emb, type_emb)


def kernel(input_ids, token_type_ids, word_emb, pos_emb, type_emb):
    b, s = input_ids.shape
    out = _embedding_sum(input_ids.astype(jnp.int32),
                         token_type_ids.astype(jnp.int32),
                         word_emb, pos_emb, type_emb)
    return out.reshape(b, s, _HIDDEN)


# speedup vs baseline: 1.0119x; 1.0119x over previous
import functools

import jax
import jax.numpy as jnp
from jax import lax
from jax.experimental import pallas as pl
from jax.experimental.pallas import tpu as pltpu
from jax.experimental.pallas import tpu_sc as plsc

_VOCAB = 100000
_HIDDEN = 128
_MAX_LEN = 2048
_BATCH = 4
_NC = 2
_NS = 16
_NW = _NC * _NS
_LANES = 16
_S_PER_W = _MAX_LEN // _NW
_TOK_PER_W = _BATCH * _S_PER_W
_JJ = _HIDDEN // _LANES


def _emb_kernel(ids_hbm, tt_hbm, word_hbm, pos_hbm, type_hbm, out_hbm,
                idx_v, tti_v, ttv_v, we_v, pe_v, ty_v,
                g_sems, out_sem, pe_sem):
    wid = lax.axis_index("s") * _NC + lax.axis_index("c")
    s0 = wid * _S_PER_W

    idx_copies = [
        pltpu.async_copy(ids_hbm.at[b, pl.ds(s0, _S_PER_W)], idx_v.at[b],
                         g_sems.at[b])
        for b in range(_BATCH)
    ]
    tt_copies = [
        pltpu.async_copy(tt_hbm.at[b, pl.ds(s0, _S_PER_W)], tti_v.at[b],
                         out_sem)
        for b in range(_BATCH)
    ]
    ty_copy = pltpu.async_copy(type_hbm, ty_v, out_sem)
    pe_copy = pltpu.async_copy(pos_hbm.at[pl.ds(s0, _S_PER_W)], pe_v, pe_sem)
    gathers = []
    for b in range(_BATCH):
        idx_copies[b].wait()
        gathers.append(
            pltpu.async_copy(word_hbm.at[idx_v.at[b]],
                             we_v.at[pl.ds(b * _S_PER_W, _S_PER_W)],
                             g_sems.at[b]))
    for c in tt_copies:
        c.wait()
    ty_copy.wait()

    def splat(g, _):
        b = g // (_S_PER_W // _LANES)
        r0 = (g % (_S_PER_W // _LANES)) * _LANES
        tts = tti_v[b, pl.ds(r0, _LANES)].astype(jnp.float32)
        for k in range(_LANES):
            ttv_v[b * _S_PER_W + r0 + k, pl.ds(0, _LANES)] = jnp.full(
                (_LANES,), tts[k], jnp.float32)
        return _

    lax.fori_loop(0, _TOK_PER_W // _LANES, splat, None)

    t0 = [ty_v[0, pl.ds(j * _LANES, _LANES)] for j in range(_JJ)]
    td = [ty_v[1, pl.ds(j * _LANES, _LANES)] - t0[j] for j in range(_JJ)]
    pe_copy.wait()

    outs = []
    for b in range(_BATCH):
        gathers[b].wait()

        def body(r, _, b=b):
            i = b * _S_PER_W + r
            ttv = ttv_v[i, pl.ds(0, _LANES)]
            for j in range(_JJ):
                c = j * _LANES
                we_v[i, pl.ds(c, _LANES)] = (
                    we_v[i, pl.ds(c, _LANES)] + pe_v[r, pl.ds(c, _LANES)]
                    + (t0[j] + ttv * td[j])
                )
            return _

        lax.fori_loop(0, _S_PER_W, body, None)
        outs.append(pltpu.async_copy(
            we_v.at[pl.ds(b * _S_PER_W, _S_PER_W)],
            out_hbm.at[pl.ds(b * _MAX_LEN + s0, _S_PER_W)],
            out_sem))
    for o in outs:
        o.wait()


@jax.jit
def _embedding_sum(ids, tt, word_emb, pos_emb, type_emb):
    mesh = plsc.VectorSubcoreMesh(core_axis_name="c", subcore_axis_name="s")
    kfn = functools.partial(
        pl.kernel,
        mesh=mesh,
        out_type=jax.ShapeDtypeStruct((_BATCH * _MAX_LEN, _HIDDEN), jnp.float32),
        scratch_types=[
            pltpu.VMEM((_BATCH, _S_PER_W), jnp.int32),
            pltpu.VMEM((_BATCH, _S_PER_W), jnp.int32),
            pltpu.VMEM((_TOK_PER_W, _LANES), jnp.float32),
            pltpu.VMEM((_TOK_PER_W, _HIDDEN), jnp.float32),
            pltpu.VMEM((_S_PER_W, _HIDDEN), jnp.float32),
            pltpu.VMEM((2, _HIDDEN), jnp.float32),
            pltpu.SemaphoreType.DMA((_BATCH,)),
            pltpu.SemaphoreType.DMA,
            pltpu.SemaphoreType.DMA,
        ],
    )(_emb_kernel)
    return kfn(ids, tt, word_emb, pos_emb, type_emb)


def kernel(input_ids, token_type_ids, word_emb, pos_emb, type_emb):
    b, s = input_ids.shape
    out = _embedding_sum(input_ids.astype(jnp.int32),
                         token_type_ids.astype(jnp.int32),
                         word_emb, pos_emb, type_emb)
    return out.reshape(b, s, _HIDDEN)


# speedup vs baseline: 1.0175x; 1.0055x over previous
import functools

import jax
import jax.numpy as jnp
from jax import lax
from jax.experimental import pallas as pl
from jax.experimental.pallas import tpu as pltpu
from jax.experimental.pallas import tpu_sc as plsc

_VOCAB = 100000
_HIDDEN = 128
_MAX_LEN = 2048
_BATCH = 4
_NC = 2
_NS = 16
_NW = _NC * _NS
_LANES = 16
_S_PER_W = _MAX_LEN // _NW
_TOK_PER_W = _BATCH * _S_PER_W
_JJ = _HIDDEN // _LANES


def _emb_kernel(ids_hbm, tt_hbm, word_hbm, pos_hbm, type_hbm, out_hbm,
                idx_v, tti_v, ttv_v, we_v, pe_v, ty_v,
                g_sems, idx_sems, tt_sem, ty_sem, pe_sem, out_sem):
    wid = lax.axis_index("s") * _NC + lax.axis_index("c")
    s0 = wid * _S_PER_W

    idx_copies = [
        pltpu.async_copy(ids_hbm.at[b, pl.ds(s0, _S_PER_W)], idx_v.at[b],
                         idx_sems.at[b])
        for b in range(_BATCH)
    ]
    tt_copies = [
        pltpu.async_copy(tt_hbm.at[b, pl.ds(s0, _S_PER_W)], tti_v.at[b],
                         tt_sem)
        for b in range(_BATCH)
    ]
    ty_copy = pltpu.async_copy(type_hbm, ty_v, ty_sem)
    pe_copy = pltpu.async_copy(pos_hbm.at[pl.ds(s0, _S_PER_W)], pe_v, pe_sem)
    gathers = []
    for b in range(_BATCH):
        idx_copies[b].wait()
        gathers.append(
            pltpu.async_copy(word_hbm.at[idx_v.at[b]],
                             we_v.at[pl.ds(b * _S_PER_W, _S_PER_W)],
                             g_sems.at[b]))
    for c in tt_copies:
        c.wait()
    ty_copy.wait()

    def splat(g, _):
        b = g // (_S_PER_W // _LANES)
        r0 = (g % (_S_PER_W // _LANES)) * _LANES
        tts = tti_v[b, pl.ds(r0, _LANES)].astype(jnp.float32)
        for k in range(_LANES):
            ttv_v[b * _S_PER_W + r0 + k, pl.ds(0, _LANES)] = jnp.full(
                (_LANES,), tts[k], jnp.float32)
        return _

    lax.fori_loop(0, _TOK_PER_W // _LANES, splat, None)

    t0 = [ty_v[0, pl.ds(j * _LANES, _LANES)] for j in range(_JJ)]
    td = [ty_v[1, pl.ds(j * _LANES, _LANES)] - t0[j] for j in range(_JJ)]
    pe_copy.wait()

    outs = []
    for b in range(_BATCH):
        gathers[b].wait()

        def body(r, _, b=b):
            i = b * _S_PER_W + r
            ttv = ttv_v[i, pl.ds(0, _LANES)]
            for j in range(_JJ):
                c = j * _LANES
                we_v[i, pl.ds(c, _LANES)] = (
                    we_v[i, pl.ds(c, _LANES)] + pe_v[r, pl.ds(c, _LANES)]
                    + (t0[j] + ttv * td[j])
                )
            return _

        lax.fori_loop(0, _S_PER_W, body, None)
        outs.append(pltpu.async_copy(
            we_v.at[pl.ds(b * _S_PER_W, _S_PER_W)],
            out_hbm.at[pl.ds(b * _MAX_LEN + s0, _S_PER_W)],
            out_sem))
    for o in outs:
        o.wait()


@jax.jit
def _embedding_sum(ids, tt, word_emb, pos_emb, type_emb):
    mesh = plsc.VectorSubcoreMesh(core_axis_name="c", subcore_axis_name="s")
    kfn = functools.partial(
        pl.kernel,
        mesh=mesh,
        out_type=jax.ShapeDtypeStruct((_BATCH * _MAX_LEN, _HIDDEN), jnp.float32),
        scratch_types=[
            pltpu.VMEM((_BATCH, _S_PER_W), jnp.int32),
            pltpu.VMEM((_BATCH, _S_PER_W), jnp.int32),
            pltpu.VMEM((_TOK_PER_W, _LANES), jnp.float32),
            pltpu.VMEM((_TOK_PER_W, _HIDDEN), jnp.float32),
            pltpu.VMEM((_S_PER_W, _HIDDEN), jnp.float32),
            pltpu.VMEM((2, _HIDDEN), jnp.float32),
            pltpu.SemaphoreType.DMA((_BATCH,)),
            pltpu.SemaphoreType.DMA((_BATCH,)),
            pltpu.SemaphoreType.DMA,
            pltpu.SemaphoreType.DMA,
            pltpu.SemaphoreType.DMA,
            pltpu.SemaphoreType.DMA,
        ],
    )(_emb_kernel)
    return kfn(ids, tt, word_emb, pos_emb, type_emb)


def kernel(input_ids, token_type_ids, word_emb, pos_emb, type_emb):
    b, s = input_ids.shape
    out = _embedding_sum(input_ids.astype(jnp.int32),
                         token_type_ids.astype(jnp.int32),
                         word_emb, pos_emb, type_emb)
    return out.reshape(b, s, _HIDDEN)
